# baseline (speedup 1.0000x reference)
import functools

import jax
import jax.numpy as jnp
from jax import lax
from jax.experimental import pallas as pl
from jax.experimental.pallas import tpu as pltpu
from jax.experimental.pallas import tpu_sc as plsc

_S = 16
_H = 128
_E = 64
_NEMO = 32
_BR = 128
_NW = 32
_CH = 128
_SL = _BR * _S


def _make_sc_gather(B):
    b_per_w = B // _NW
    nch = b_per_w // _CH
    mesh = plsc.VectorSubcoreMesh(core_axis_name="c", subcore_axis_name="s")

    @functools.partial(
        pl.kernel,
        out_type=jax.ShapeDtypeStruct((B, _H), jnp.float32),
        mesh=mesh,
        scratch_types=[
            [pltpu.VMEM((_CH,), jnp.int32) for _ in range(nch)],
            pltpu.VMEM((b_per_w, _H), jnp.float32),
            pltpu.SemaphoreType.DMA,
            pltpu.SemaphoreType.DMA,
        ],
    )
    def gather(states_hbm, flat_hbm, out_hbm, idx_vs, rows_v, sem_i, sem_g):
        wid = lax.axis_index("s") * 2 + lax.axis_index("c")
        base = wid * b_per_w
        idx_cps = [
            pltpu.async_copy(flat_hbm.at[pl.ds(base + j * _CH, _CH)],
                             idx_vs[j], sem_i)
            for j in range(nch)
        ]
        for cp in idx_cps:
            cp.wait()
        row_cps = [
            pltpu.async_copy(states_hbm.at[idx_vs[j]],
                             rows_v.at[pl.ds(j * _CH, _CH)], sem_g)
            for j in range(nch)
        ]
        for cp in row_cps:
            cp.wait()
        pltpu.sync_copy(rows_v, out_hbm.at[pl.ds(base, b_per_w)])

    return gather


def _mega_body(emo_ref, hold_ref, mcol_ref, states_ref, embed_ref, wih_ref,
               whh_ref, bih_ref, bhh_ref, out_ref):
    emo = emo_ref[...]
    h_old = hold_ref[...]

    safe = jnp.where(emo >= 0, emo, _NEMO)
    cols = lax.broadcasted_iota(jnp.int32, (1, _NEMO + 1), 1)
    onehot = (safe == cols).astype(jnp.float32)
    emb = jnp.dot(onehot, embed_ref[...],
                  preferred_element_type=jnp.float32)
    gi = jnp.dot(emb, wih_ref[...],
                 preferred_element_type=jnp.float32) + bih_ref[...]
    gh = jnp.dot(h_old, whh_ref[...],
                 preferred_element_type=jnp.float32) + bhh_ref[...]
    r = jax.nn.sigmoid(gi[:, :_H] + gh[:, :_H])
    z = jax.nn.sigmoid(gi[:, _H:2 * _H] + gh[:, _H:2 * _H])
    n = jnp.tanh(gi[:, 2 * _H:] + r * gh[:, 2 * _H:])
    h_new = (1.0 - z) * n + z * h_old

    repl = (lax.broadcasted_iota(jnp.int32, (_SL, _BR), 0) // _S ==
            lax.broadcasted_iota(jnp.int32, (_SL, _BR), 1))
    hnew_rep = jnp.dot(repl.astype(jnp.float32), h_new,
                       preferred_element_type=jnp.float32)
    m_lanes = jnp.dot(mcol_ref[...], jnp.ones((1, _H), jnp.float32),
                      preferred_element_type=jnp.float32)
    x = states_ref[...]
    out_ref[...] = (x - m_lanes * x) + m_lanes * hnew_rep


def _tc_mega(states_flat, h_old, emo, mcol, embed, wih_t, whh_t, bih, bhh):
    BS, H = states_flat.shape
    B = BS // _S
    nb = B // _BR
    return pl.pallas_call(
        _mega_body,
        grid=(nb,),
        in_specs=[
            pl.BlockSpec((_BR, 1), lambda i: (i, 0)),
            pl.BlockSpec((_BR, _H), lambda i: (i, 0)),
            pl.BlockSpec((_SL, 1), lambda i: (i, 0)),
            pl.BlockSpec((_SL, _H), lambda i: (i, 0)),
            pl.BlockSpec((_NEMO + 1, _E), lambda i: (0, 0)),
            pl.BlockSpec((_E, 3 * _H), lambda i: (0, 0)),
            pl.BlockSpec((_H, 3 * _H), lambda i: (0, 0)),
            pl.BlockSpec((1, 3 * _H), lambda i: (0, 0)),
            pl.BlockSpec((1, 3 * _H), lambda i: (0, 0)),
        ],
        out_specs=pl.BlockSpec((_SL, _H), lambda i: (i, 0)),
        out_shape=jax.ShapeDtypeStruct((BS, H), jnp.float32),
        compiler_params=pltpu.CompilerParams(
            dimension_semantics=("arbitrary",),
        ),
    )(emo, h_old, mcol, states_flat, embed, wih_t, whh_t, bih, bhh)


def kernel(states, speaker_ids, emotion_ids, embed, W_ih, W_hh, b_ih, b_hh):
    B, S, H = states.shape
    idx = jnp.minimum(speaker_ids.astype(jnp.int32), S - 1)
    flat = jnp.arange(B, dtype=jnp.int32) * S + idx
    emo = emotion_ids.astype(jnp.int32).reshape(B, 1)
    mcol = (idx[:, None] == jnp.arange(S, dtype=jnp.int32)[None, :]
            ).astype(jnp.float32).reshape(B * S, 1)
    states_flat = states.reshape(B * S, H)

    h_old = _make_sc_gather(B)(states_flat, flat)
    out = _tc_mega(states_flat, h_old, emo, mcol, embed,
                   W_ih.T, W_hh.T,
                   b_ih.reshape(1, -1), b_hh.reshape(1, -1))
    return out.reshape(B, S, H)


# speedup vs baseline: 2.4330x; 2.4330x over previous
import functools

import jax
import jax.numpy as jnp
from jax import lax
from jax.experimental import pallas as pl
from jax.experimental.pallas import tpu as pltpu
from jax.experimental.pallas import tpu_sc as plsc

_S = 16
_H = 128
_E = 64
_NEMO = 32
_BR = 512
_NW = 32
_CH = 128


def _sc_mesh():
    return plsc.VectorSubcoreMesh(core_axis_name="c", subcore_axis_name="s")


def _make_sc_gather(B):
    b_per_w = B // _NW
    nch = b_per_w // _CH

    @functools.partial(
        pl.kernel,
        out_type=jax.ShapeDtypeStruct((B, _H), jnp.float32),
        mesh=_sc_mesh(),
        scratch_types=[
            [pltpu.VMEM((_CH,), jnp.int32) for _ in range(nch)],
            pltpu.VMEM((b_per_w, _H), jnp.float32),
            pltpu.SemaphoreType.DMA,
            pltpu.SemaphoreType.DMA,
        ],
    )
    def gather(states_hbm, flat_hbm, out_hbm, idx_vs, rows_v, sem_i, sem_g):
        wid = lax.axis_index("s") * 2 + lax.axis_index("c")
        base = wid * b_per_w
        idx_cps = [
            pltpu.async_copy(flat_hbm.at[pl.ds(base + j * _CH, _CH)],
                             idx_vs[j], sem_i)
            for j in range(nch)
        ]
        for cp in idx_cps:
            cp.wait()
        row_cps = [
            pltpu.async_copy(states_hbm.at[idx_vs[j]],
                             rows_v.at[pl.ds(j * _CH, _CH)], sem_g)
            for j in range(nch)
        ]
        for cp in row_cps:
            cp.wait()
        pltpu.sync_copy(rows_v, out_hbm.at[pl.ds(base, b_per_w)])

    return gather


def _make_sc_scatter(B):
    b_per_w = B // _NW
    nch = b_per_w // _CH

    @functools.partial(
        pl.kernel,
        out_type=(),
        mesh=_sc_mesh(),
        scratch_types=[
            [pltpu.VMEM((_CH,), jnp.int32) for _ in range(nch)],
            pltpu.VMEM((b_per_w, _H), jnp.float32),
            pltpu.SemaphoreType.DMA,
            pltpu.SemaphoreType.DMA,
        ],
    )
    def scatter(out_ref, flat_hbm, hnew_hbm, idx_vs, rows_v, sem_i, sem_s):
        wid = lax.axis_index("s") * 2 + lax.axis_index("c")
        base = wid * b_per_w
        idx_cps = [
            pltpu.async_copy(flat_hbm.at[pl.ds(base + j * _CH, _CH)],
                             idx_vs[j], sem_i)
            for j in range(nch)
        ]
        rows_cp = pltpu.async_copy(hnew_hbm.at[pl.ds(base, b_per_w)],
                                   rows_v, sem_s)
        for cp in idx_cps:
            cp.wait()
        rows_cp.wait()
        out_cps = [
            pltpu.async_copy(rows_v.at[pl.ds(j * _CH, _CH)],
                             out_ref.at[idx_vs[j]], sem_s)
            for j in range(nch)
        ]
        for cp in out_cps:
            cp.wait()

    return scatter


def _copy_gru_body(emo_ref, hold_ref, states_ref, embed_ref, wih_ref,
                   whh_ref, bih_ref, bhh_ref, out_ref, hnew_ref):
    out_ref[...] = states_ref[...]

    emo = emo_ref[...]
    h_old = hold_ref[...]
    safe = jnp.where(emo >= 0, emo, _NEMO)
    cols = lax.broadcasted_iota(jnp.int32, (1, _NEMO + 1), 1)
    onehot = (safe == cols).astype(jnp.float32)
    emb = jnp.dot(onehot, embed_ref[...],
                  preferred_element_type=jnp.float32)
    gi = jnp.dot(emb, wih_ref[...],
                 preferred_element_type=jnp.float32) + bih_ref[...]
    gh = jnp.dot(h_old, whh_ref[...],
                 preferred_element_type=jnp.float32) + bhh_ref[...]
    r = jax.nn.sigmoid(gi[:, :_H] + gh[:, :_H])
    z = jax.nn.sigmoid(gi[:, _H:2 * _H] + gh[:, _H:2 * _H])
    n = jnp.tanh(gi[:, 2 * _H:] + r * gh[:, 2 * _H:])
    hnew_ref[...] = (1.0 - z) * n + z * h_old


def _tc_copy_gru(states_flat, h_old, emo, embed, wih_t, whh_t, bih, bhh):
    BS, H = states_flat.shape
    B = BS // _S
    nb = B // _BR
    rows = _BR * _S
    return pl.pallas_call(
        _copy_gru_body,
        grid=(nb,),
        in_specs=[
            pl.BlockSpec((_BR, 1), lambda i: (i, 0)),
            pl.BlockSpec((_BR, _H), lambda i: (i, 0)),
            pl.BlockSpec((rows, _H), lambda i: (i, 0)),
            pl.BlockSpec((_NEMO + 1, _E), lambda i: (0, 0)),
            pl.BlockSpec((_E, 3 * _H), lambda i: (0, 0)),
            pl.BlockSpec((_H, 3 * _H), lambda i: (0, 0)),
            pl.BlockSpec((1, 3 * _H), lambda i: (0, 0)),
            pl.BlockSpec((1, 3 * _H), lambda i: (0, 0)),
        ],
        out_specs=[
            pl.BlockSpec((rows, _H), lambda i: (i, 0)),
            pl.BlockSpec((_BR, _H), lambda i: (i, 0)),
        ],
        out_shape=[
            jax.ShapeDtypeStruct((BS, H), jnp.float32),
            jax.ShapeDtypeStruct((B, _H), jnp.float32),
        ],
        compiler_params=pltpu.CompilerParams(
            dimension_semantics=("arbitrary",),
        ),
    )(emo, h_old, states_flat, embed, wih_t, whh_t, bih, bhh)


def kernel(states, speaker_ids, emotion_ids, embed, W_ih, W_hh, b_ih, b_hh):
    B, S, H = states.shape
    idx = jnp.minimum(speaker_ids.astype(jnp.int32), S - 1)
    flat = jnp.arange(B, dtype=jnp.int32) * S + idx
    emo = emotion_ids.astype(jnp.int32).reshape(B, 1)
    states_flat = states.reshape(B * S, H)

    h_old = _make_sc_gather(B)(states_flat, flat)
    out0, h_new = _tc_copy_gru(states_flat, h_old, emo, embed,
                               W_ih.T, W_hh.T,
                               b_ih.reshape(1, -1), b_hh.reshape(1, -1))
    out_ref = jax.new_ref(out0)
    _make_sc_scatter(B)(out_ref, flat, h_new)
    return out_ref[...].reshape(B, S, H)


# speedup vs baseline: 2.4723x; 1.0161x over previous
import functools

import jax
import jax.numpy as jnp
from jax import lax
from jax.experimental import pallas as pl
from jax.experimental.pallas import tpu as pltpu
from jax.experimental.pallas import tpu_sc as plsc

_S = 16
_H = 128
_E = 64
_NEMO = 32
_BR = 1024
_NW = 32
_CH = 128


def _sc_mesh():
    return plsc.VectorSubcoreMesh(core_axis_name="c", subcore_axis_name="s")


def _make_sc_gather(B):
    b_per_w = B // _NW
    nch = b_per_w // _CH

    @functools.partial(
        pl.kernel,
        out_type=jax.ShapeDtypeStruct((B, _H), jnp.float32),
        mesh=_sc_mesh(),
        scratch_types=[
            [pltpu.VMEM((_CH,), jnp.int32) for _ in range(nch)],
            pltpu.VMEM((b_per_w, _H), jnp.float32),
            pltpu.SemaphoreType.DMA,
            pltpu.SemaphoreType.DMA,
        ],
    )
    def gather(states_hbm, flat_hbm, out_hbm, idx_vs, rows_v, sem_i, sem_g):
        wid = lax.axis_index("s") * 2 + lax.axis_index("c")
        base = wid * b_per_w
        idx_cps = [
            pltpu.async_copy(flat_hbm.at[pl.ds(base + j * _CH, _CH)],
                             idx_vs[j], sem_i)
            for j in range(nch)
        ]
        for cp in idx_cps:
            cp.wait()
        row_cps = [
            pltpu.async_copy(states_hbm.at[idx_vs[j]],
                             rows_v.at[pl.ds(j * _CH, _CH)], sem_g)
            for j in range(nch)
        ]
        for cp in row_cps:
            cp.wait()
        pltpu.sync_copy(rows_v, out_hbm.at[pl.ds(base, b_per_w)])

    return gather


def _make_sc_scatter(B):
    b_per_w = B // _NW
    nch = b_per_w // _CH

    @functools.partial(
        pl.kernel,
        out_type=(),
        mesh=_sc_mesh(),
        scratch_types=[
            [pltpu.VMEM((_CH,), jnp.int32) for _ in range(nch)],
            pltpu.VMEM((b_per_w, _H), jnp.float32),
            pltpu.SemaphoreType.DMA,
            pltpu.SemaphoreType.DMA,
        ],
    )
    def scatter(out_ref, flat_hbm, hnew_hbm, idx_vs, rows_v, sem_i, sem_s):
        wid = lax.axis_index("s") * 2 + lax.axis_index("c")
        base = wid * b_per_w
        idx_cps = [
            pltpu.async_copy(flat_hbm.at[pl.ds(base + j * _CH, _CH)],
                             idx_vs[j], sem_i)
            for j in range(nch)
        ]
        rows_cp = pltpu.async_copy(hnew_hbm.at[pl.ds(base, b_per_w)],
                                   rows_v, sem_s)
        for cp in idx_cps:
            cp.wait()
        rows_cp.wait()
        out_cps = [
            pltpu.async_copy(rows_v.at[pl.ds(j * _CH, _CH)],
                             out_ref.at[idx_vs[j]], sem_s)
            for j in range(nch)
        ]
        for cp in out_cps:
            cp.wait()

    return scatter


def _copy_gru_body(emo_ref, hold_ref, states_ref, embed_ref, wih_ref,
                   whh_ref, bih_ref, bhh_ref, out_ref, hnew_ref):
    out_ref[...] = states_ref[...]

    emo = emo_ref[...]
    h_old = hold_ref[...]
    safe = jnp.where(emo >= 0, emo, _NEMO)
    cols = lax.broadcasted_iota(jnp.int32, (1, _NEMO + 1), 1)
    onehot = (safe == cols).astype(jnp.float32)
    emb = jnp.dot(onehot, embed_ref[...],
                  preferred_element_type=jnp.float32)
    gi = jnp.dot(emb, wih_ref[...],
                 preferred_element_type=jnp.float32) + bih_ref[...]
    gh = jnp.dot(h_old, whh_ref[...],
                 preferred_element_type=jnp.float32) + bhh_ref[...]
    r = jax.nn.sigmoid(gi[:, :_H] + gh[:, :_H])
    z = jax.nn.sigmoid(gi[:, _H:2 * _H] + gh[:, _H:2 * _H])
    n = jnp.tanh(gi[:, 2 * _H:] + r * gh[:, 2 * _H:])
    hnew_ref[...] = (1.0 - z) * n + z * h_old


def _tc_copy_gru(states_flat, h_old, emo, embed, wih_t, whh_t, bih, bhh):
    BS, H = states_flat.shape
    B = BS // _S
    nb = B // _BR
    rows = _BR * _S
    return pl.pallas_call(
        _copy_gru_body,
        grid=(nb,),
        in_specs=[
            pl.BlockSpec((_BR, 1), lambda i: (i, 0)),
            pl.BlockSpec((_BR, _H), lambda i: (i, 0)),
            pl.BlockSpec((rows, _H), lambda i: (i, 0)),
            pl.BlockSpec((_NEMO + 1, _E), lambda i: (0, 0)),
            pl.BlockSpec((_E, 3 * _H), lambda i: (0, 0)),
            pl.BlockSpec((_H, 3 * _H), lambda i: (0, 0)),
            pl.BlockSpec((1, 3 * _H), lambda i: (0, 0)),
            pl.BlockSpec((1, 3 * _H), lambda i: (0, 0)),
        ],
        out_specs=[
            pl.BlockSpec((rows, _H), lambda i: (i, 0)),
            pl.BlockSpec((_BR, _H), lambda i: (i, 0)),
        ],
        out_shape=[
            jax.ShapeDtypeStruct((BS, H), jnp.float32),
            jax.ShapeDtypeStruct((B, _H), jnp.float32),
        ],
        compiler_params=pltpu.CompilerParams(
            dimension_semantics=("arbitrary",),
        ),
    )(emo, h_old, states_flat, embed, wih_t, whh_t, bih, bhh)


def kernel(states, speaker_ids, emotion_ids, embed, W_ih, W_hh, b_ih, b_hh):
    B, S, H = states.shape
    idx = jnp.minimum(speaker_ids.astype(jnp.int32), S - 1)
    flat = jnp.arange(B, dtype=jnp.int32) * S + idx
    emo = emotion_ids.astype(jnp.int32).reshape(B, 1)
    states_flat = states.reshape(B * S, H)

    h_old = _make_sc_gather(B)(states_flat, flat)
    out0, h_new = _tc_copy_gru(states_flat, h_old, emo, embed,
                               W_ih.T, W_hh.T,
                               b_ih.reshape(1, -1), b_hh.reshape(1, -1))
    out_ref = jax.new_ref(out0)
    _make_sc_scatter(B)(out_ref, flat, h_new)
    return out_ref[...].reshape(B, S, H)
